# baseline (speedup 1.0000x reference)
import functools

import jax
import jax.numpy as jnp
from jax import lax
from jax.experimental import pallas as pl
from jax.experimental.pallas import tpu as pltpu
from jax.experimental.pallas import tpu_sc as plsc

EMB = 300
BATCH = 4096
WIN = 50
WINP = 56
NC, NS = 2, 16
NW = NC * NS
BPW = BATCH // NW
DMAIN = 256
DTAIL = EMB - DMAIN
DTAILP = 128
DTACC = 48


def _gather_src(e_hbm, idx_v, c, ncols):
    if ncols == e_hbm.shape[1]:
        return e_hbm.at[idx_v.at[c]]
    return e_hbm.at[idx_v.at[c], pl.ds(0, ncols)]


def _worker_id():
    return lax.axis_index("s") * NC + lax.axis_index("c")


def _acc_add(ref_slice, x):
    plsc.addupdate(ref_slice, x)


def _make_pool_body(ncols, nacc):
    def body(e_hbm, idx_hbm, out_hbm, idx_v, acc_v, buf_a, buf_b,
             sem_a, sem_b):
        w = _worker_id()
        pltpu.sync_copy(idx_hbm.at[w], idx_v)

        zero = jnp.zeros((16,), jnp.float32)
        nzero = ncols // 16
        nvec = nacc // 16

        def _zero_body(i, _):
            for k in range(nzero):
                acc_v[i, pl.ds(k * 16, 16)] = zero
            return 0

        lax.fori_loop(0, BPW, _zero_body, 0, unroll=False)

        def _fire(c, buf, sem):
            return pltpu.async_copy(_gather_src(e_hbm, idx_v, c, ncols),
                                    buf, sem)

        def _wait(c, buf, sem):
            pltpu.make_async_copy(_gather_src(e_hbm, idx_v, c, ncols),
                                  buf, sem).wait()

        def _accum(buf):
            def _add_body(i, _):
                for k in range(nvec):
                    sl = pl.ds(k * 16, 16)
                    _acc_add(acc_v.at[i, sl], buf[i, sl])
                return 0

            lax.fori_loop(0, BPW, _add_body, 0, unroll=False)

        _fire(0, buf_a, sem_a)
        _fire(1, buf_b, sem_b)

        def _chunk_body(i, _):
            c = 2 * i
            _wait(c, buf_a, sem_a)
            _accum(buf_a)

            @pl.when(c + 2 < WIN)
            def _():
                _fire(c + 2, buf_a, sem_a)

            _wait(c + 1, buf_b, sem_b)
            _accum(buf_b)

            @pl.when(c + 3 < WIN)
            def _():
                _fire(c + 3, buf_b, sem_b)

            return 0

        lax.fori_loop(0, (WIN + 1) // 2, _chunk_body, 0, unroll=False)

        pltpu.sync_copy(acc_v, out_hbm.at[w])

    return body


def _make_pool(ncols, nacc):
    mesh = plsc.VectorSubcoreMesh(core_axis_name="c", subcore_axis_name="s",
                                  num_cores=NC, num_subcores=NS)
    return pl.kernel(
        _make_pool_body(ncols, nacc),
        out_type=jax.ShapeDtypeStruct((NW, BPW, ncols), jnp.float32),
        mesh=mesh,
        scratch_types=[
            pltpu.VMEM((WINP, BPW), jnp.int32),
            pltpu.VMEM((BPW, ncols), jnp.float32),
            pltpu.VMEM((BPW, ncols), jnp.float32),
            pltpu.VMEM((BPW, ncols), jnp.float32),
            pltpu.SemaphoreType.DMA,
            pltpu.SemaphoreType.DMA,
        ],
        name=f"embed_pool_sc_{ncols}",
    )


@jax.jit
def _pool_main(E, idx_grp):
    return _make_pool(DMAIN, DMAIN)(E, idx_grp)


@jax.jit
def _pool_tail(et, idx_grp):
    return _make_pool(DTAILP, DTACC)(et, idx_grp)


def _tail_table_kernel(e_ref, out_ref):
    out_ref[:, 0:DTAIL] = e_ref[:, DMAIN:EMB]
    out_ref[:, DTAIL:] = jnp.zeros_like(out_ref[:, DTAIL:])


def _tail_table(E):
    RB = 8192
    v = E.shape[0]
    grid = ((v + RB - 1) // RB,)
    return pl.pallas_call(
        _tail_table_kernel,
        grid=grid,
        in_specs=[pl.BlockSpec((RB, EMB), lambda i: (i, 0))],
        out_specs=pl.BlockSpec((RB, DTAILP), lambda i: (i, 0)),
        out_shape=jax.ShapeDtypeStruct((v, DTAILP), jnp.float32),
    )(E)


def _leaky(x):
    return jnp.where(x >= 0, x, 0.3 * x)


def _mlp_kernel(x_ref, w1_ref, b1_ref, w2_ref, b2_ref, wo_ref, bo_ref,
                out_ref):
    x = x_ref[...]
    h1 = _leaky(jnp.dot(x, w1_ref[...],
                        preferred_element_type=jnp.float32) + b1_ref[...])
    h2 = _leaky(jnp.dot(h1, w2_ref[...],
                        preferred_element_type=jnp.float32) + b2_ref[...])
    logits = jnp.dot(h2, wo_ref[...],
                     preferred_element_type=jnp.float32) + bo_ref[...]
    z = logits - jnp.max(logits, axis=-1, keepdims=True)
    e = jnp.exp(z)
    out_ref[...] = e / jnp.sum(e, axis=-1, keepdims=True)


def _mlp(pooled, W1, b1, W2, b2, Wout, bout):
    BB = 512
    grid = (BATCH // BB,)
    h1, h2, no = W1.shape[1], W2.shape[1], Wout.shape[1]
    return pl.pallas_call(
        _mlp_kernel,
        grid=grid,
        in_specs=[
            pl.BlockSpec((BB, EMB), lambda i: (i, 0)),
            pl.BlockSpec((EMB, h1), lambda i: (0, 0)),
            pl.BlockSpec((1, h1), lambda i: (0, 0)),
            pl.BlockSpec((h1, h2), lambda i: (0, 0)),
            pl.BlockSpec((1, h2), lambda i: (0, 0)),
            pl.BlockSpec((h2, no), lambda i: (0, 0)),
            pl.BlockSpec((1, no), lambda i: (0, 0)),
        ],
        out_specs=pl.BlockSpec((BB, no), lambda i: (i, 0)),
        out_shape=jax.ShapeDtypeStruct((BATCH, no), jnp.float32),
    )(pooled, W1, b1.reshape(1, h1), W2, b2.reshape(1, h2),
      Wout, bout.reshape(1, no))


def kernel(inputs, E, W1, b1, W2, b2, Wout, bout):
    idx = inputs.astype(jnp.int32)
    idx_grp = jnp.pad(idx.reshape(NW, BPW, WIN).transpose(0, 2, 1),
                      ((0, 0), (0, WINP - WIN), (0, 0)))
    et = _tail_table(E)
    p_main = _pool_main(E, idx_grp).reshape(BATCH, DMAIN)
    p_tail = _pool_tail(et, idx_grp).reshape(BATCH, DTAILP)[:, :DTAIL]
    pooled = jnp.concatenate([p_main, p_tail], axis=1)
    return _mlp(pooled, W1, b1, W2, b2, Wout, bout)


# speedup vs baseline: 1.4386x; 1.4386x over previous
import functools

import jax
import jax.numpy as jnp
from jax import lax
from jax.experimental import pallas as pl
from jax.experimental.pallas import tpu as pltpu
from jax.experimental.pallas import tpu_sc as plsc

EMB = 300
BATCH = 4096
WIN = 50
WINP = 56
NC, NS = 2, 16
NW = NC * NS
BPW = BATCH // NW
DMAIN = 256
DTAIL = EMB - DMAIN
DTAILP = 128


def _gather_src(e_hbm, idx_v, c, ncols):
    if ncols == e_hbm.shape[1]:
        return e_hbm.at[idx_v.at[c]]
    return e_hbm.at[idx_v.at[c], pl.ds(0, ncols)]


def _worker_id():
    return lax.axis_index("s") * NC + lax.axis_index("c")


def _acc_add(ref_slice, x):
    plsc.addupdate(ref_slice, x)


def _make_pool_body(ncols, nacc):
    def body(e_hbm, idx_hbm, out_hbm, idx_v, acc_v, buf_a, buf_b,
             sem_a, sem_b):
        w = _worker_id()
        pltpu.sync_copy(idx_hbm.at[w], idx_v)

        zero = jnp.zeros((16,), jnp.float32)
        nzero = ncols // 16
        nvec = nacc // 16

        def _zero_body(i, _):
            for k in range(nzero):
                acc_v[i, pl.ds(k * 16, 16)] = zero
            return 0

        lax.fori_loop(0, BPW, _zero_body, 0, unroll=False)

        def _fire(c, buf, sem):
            return pltpu.async_copy(_gather_src(e_hbm, idx_v, c, ncols),
                                    buf, sem)

        def _wait(c, buf, sem):
            pltpu.make_async_copy(_gather_src(e_hbm, idx_v, c, ncols),
                                  buf, sem).wait()

        def _accum(buf):
            def _add_body(i, _):
                for k in range(nvec):
                    sl = pl.ds(k * 16, 16)
                    _acc_add(acc_v.at[i, sl], buf[i, sl])
                return 0

            lax.fori_loop(0, BPW, _add_body, 0, unroll=False)

        _fire(0, buf_a, sem_a)
        _fire(1, buf_b, sem_b)

        def _chunk_body(i, _):
            c = 2 * i
            _wait(c, buf_a, sem_a)
            _accum(buf_a)

            @pl.when(c + 2 < WIN)
            def _():
                _fire(c + 2, buf_a, sem_a)

            _wait(c + 1, buf_b, sem_b)
            _accum(buf_b)

            @pl.when(c + 3 < WIN)
            def _():
                _fire(c + 3, buf_b, sem_b)

            return 0

        lax.fori_loop(0, (WIN + 1) // 2, _chunk_body, 0, unroll=False)

        pltpu.sync_copy(acc_v, out_hbm.at[w])

    return body


def _make_pool(ncols, nacc):
    mesh = plsc.VectorSubcoreMesh(core_axis_name="c", subcore_axis_name="s",
                                  num_cores=NC, num_subcores=NS)
    return pl.kernel(
        _make_pool_body(ncols, nacc),
        out_type=jax.ShapeDtypeStruct((NW, BPW, ncols), jnp.float32),
        mesh=mesh,
        scratch_types=[
            pltpu.VMEM((WINP, BPW), jnp.int32),
            pltpu.VMEM((BPW, ncols), jnp.float32),
            pltpu.VMEM((BPW, ncols), jnp.float32),
            pltpu.VMEM((BPW, ncols), jnp.float32),
            pltpu.SemaphoreType.DMA,
            pltpu.SemaphoreType.DMA,
        ],
        name=f"embed_pool_sc_{ncols}",
    )


@jax.jit
def _pool_main(E, idx_grp):
    return _make_pool(DMAIN, DMAIN)(E, idx_grp)


@jax.jit
def _pool_tail(E, idx_grp):
    mesh = plsc.VectorSubcoreMesh(core_axis_name="c", subcore_axis_name="s",
                                  num_cores=NC, num_subcores=NS)
    return pl.kernel(
        _tail_body,
        out_type=jax.ShapeDtypeStruct((NW, BPW, DTAILP), jnp.float32),
        mesh=mesh,
        scratch_types=[
            pltpu.VMEM((WINP, BPW), jnp.int32),
            pltpu.VMEM((BPW, DTAILP), jnp.float32),
            pltpu.VMEM((BPW, DTAIL), jnp.float32),
            pltpu.VMEM((BPW, DTAIL), jnp.float32),
            pltpu.SemaphoreType.DMA,
            pltpu.SemaphoreType.DMA,
        ],
        name="embed_pool_sc_tail",
    )(E, idx_grp)


def _tail_body(e_hbm, idx_hbm, out_hbm, idx_v, acc_v, buf_a, buf_b,
               sem_a, sem_b):
    w = _worker_id()
    pltpu.sync_copy(idx_hbm.at[w], idx_v)

    zero = jnp.zeros((16,), jnp.float32)
    ovl = DTAIL - 16
    ovl_mask = lax.iota(jnp.int32, 16) < (32 - ovl)

    def _zero_body(i, _):
        for k in range(3):
            acc_v[i, pl.ds(k * 16, 16)] = zero
        return 0

    lax.fori_loop(0, BPW, _zero_body, 0, unroll=False)

    def _fire(c, buf, sem):
        def _fire_body(t16, _):
            v = idx_v[c, pl.ds(t16 * 16, 16)]
            for l in range(16):
                r = v[l]
                pltpu.async_copy(
                    e_hbm.at[pl.ds(r, 1), pl.ds(DMAIN, DTAIL)],
                    buf.at[pl.ds(t16 * 16 + l, 1)], sem)
            return 0

        lax.fori_loop(0, BPW // 16, _fire_body, 0, unroll=False)

    def _wait_all(buf, sem):
        pltpu.make_async_copy(e_hbm.at[pl.ds(0, BPW), pl.ds(DMAIN, DTAIL)],
                              buf, sem).wait()

    def _accum(buf):
        def _add_body(i, _):
            for k in range(2):
                sl = pl.ds(k * 16, 16)
                _acc_add(acc_v.at[i, sl], buf[i, sl])
            t = buf[i, pl.ds(ovl, 16)]
            t = jnp.where(ovl_mask, 0.0, t)
            _acc_add(acc_v.at[i, pl.ds(ovl, 16)], t)
            return 0

        lax.fori_loop(0, BPW, _add_body, 0, unroll=False)

    _fire(0, buf_a, sem_a)
    _fire(1, buf_b, sem_b)

    def _chunk_body(i, _):
        c = 2 * i
        _wait_all(buf_a, sem_a)
        _accum(buf_a)

        @pl.when(c + 2 < WIN)
        def _():
            _fire(c + 2, buf_a, sem_a)

        _wait_all(buf_b, sem_b)
        _accum(buf_b)

        @pl.when(c + 3 < WIN)
        def _():
            _fire(c + 3, buf_b, sem_b)

        return 0

    lax.fori_loop(0, (WIN + 1) // 2, _chunk_body, 0, unroll=False)

    pltpu.sync_copy(acc_v, out_hbm.at[w])


def _leaky(x):
    return jnp.where(x >= 0, x, 0.3 * x)


def _mlp_kernel(x_ref, w1_ref, b1_ref, w2_ref, b2_ref, wo_ref, bo_ref,
                out_ref):
    x = x_ref[...]
    h1 = _leaky(jnp.dot(x, w1_ref[...],
                        preferred_element_type=jnp.float32) + b1_ref[...])
    h2 = _leaky(jnp.dot(h1, w2_ref[...],
                        preferred_element_type=jnp.float32) + b2_ref[...])
    logits = jnp.dot(h2, wo_ref[...],
                     preferred_element_type=jnp.float32) + bo_ref[...]
    z = logits - jnp.max(logits, axis=-1, keepdims=True)
    e = jnp.exp(z)
    out_ref[...] = e / jnp.sum(e, axis=-1, keepdims=True)


def _mlp(pooled, W1, b1, W2, b2, Wout, bout):
    BB = 512
    grid = (BATCH // BB,)
    h1, h2, no = W1.shape[1], W2.shape[1], Wout.shape[1]
    return pl.pallas_call(
        _mlp_kernel,
        grid=grid,
        in_specs=[
            pl.BlockSpec((BB, EMB), lambda i: (i, 0)),
            pl.BlockSpec((EMB, h1), lambda i: (0, 0)),
            pl.BlockSpec((1, h1), lambda i: (0, 0)),
            pl.BlockSpec((h1, h2), lambda i: (0, 0)),
            pl.BlockSpec((1, h2), lambda i: (0, 0)),
            pl.BlockSpec((h2, no), lambda i: (0, 0)),
            pl.BlockSpec((1, no), lambda i: (0, 0)),
        ],
        out_specs=pl.BlockSpec((BB, no), lambda i: (i, 0)),
        out_shape=jax.ShapeDtypeStruct((BATCH, no), jnp.float32),
    )(pooled, W1, b1.reshape(1, h1), W2, b2.reshape(1, h2),
      Wout, bout.reshape(1, no))


def kernel(inputs, E, W1, b1, W2, b2, Wout, bout):
    idx = inputs.astype(jnp.int32)
    idx_grp = jnp.pad(idx.reshape(NW, BPW, WIN).transpose(0, 2, 1),
                      ((0, 0), (0, WINP - WIN), (0, 0)))
    p_main = _pool_main(E, idx_grp).reshape(BATCH, DMAIN)
    p_tail = _pool_tail(E, idx_grp).reshape(BATCH, DTAILP)[:, :DTAIL]
    pooled = jnp.concatenate([p_main, p_tail], axis=1)
    return _mlp(pooled, W1, b1, W2, b2, Wout, bout)
